# baseline (speedup 1.0000x reference)
import jax
import jax.numpy as jnp
from jax.experimental import pallas as pl
from jax.experimental.pallas import tpu as pltpu

_TS = 1024
_BB = 2


def _fused_kernel(table_ref, w_ref, bias_ref, x_ref, o_ref, pos_ref):
    bidx = pl.program_id(1)

    @pl.when(bidx == 0)
    def _():
        pos_ref[...] = (
            jnp.dot(table_ref[...].astype(jnp.bfloat16),
                    w_ref[...].astype(jnp.bfloat16),
                    preferred_element_type=jnp.float32)
            + bias_ref[...]
        )

    o_ref[...] = x_ref[...] + pos_ref[...][None, :, :]


def kernel(inputs, table, W, b):
    B, S, D = inputs.shape
    bias2d = b.reshape(1, D)
    grid = (S // _TS, B // _BB)
    return pl.pallas_call(
        _fused_kernel,
        grid=grid,
        in_specs=[
            pl.BlockSpec((_TS, D), lambda s, bb: (s, 0)),
            pl.BlockSpec((D, D), lambda s, bb: (0, 0)),
            pl.BlockSpec((1, D), lambda s, bb: (0, 0)),
            pl.BlockSpec((_BB, _TS, D), lambda s, bb: (bb, s, 0)),
        ],
        out_specs=pl.BlockSpec((_BB, _TS, D), lambda s, bb: (bb, s, 0)),
        out_shape=jax.ShapeDtypeStruct((B, S, D), jnp.float32),
        scratch_shapes=[pltpu.VMEM((_TS, D), jnp.float32)],
    )(table, W, bias2d, inputs)


# speedup vs baseline: 1.0805x; 1.0805x over previous
import jax
import jax.numpy as jnp
from jax.experimental import pallas as pl
from jax.experimental.pallas import tpu as pltpu

_TS = 256


def _fused_kernel(table_ref, w_ref, bias_ref, x_ref, o_ref):
    pos = (
        jnp.dot(table_ref[...].astype(jnp.bfloat16),
                w_ref[...].astype(jnp.bfloat16),
                preferred_element_type=jnp.float32)
        + bias_ref[...]
    )
    o_ref[...] = x_ref[...] + pos[None, :, :]


def kernel(inputs, table, W, b):
    B, S, D = inputs.shape
    bias2d = b.reshape(1, D)
    grid = (S // _TS,)
    return pl.pallas_call(
        _fused_kernel,
        grid=grid,
        in_specs=[
            pl.BlockSpec((_TS, D), lambda s: (s, 0)),
            pl.BlockSpec((D, D), lambda s: (0, 0)),
            pl.BlockSpec((1, D), lambda s: (0, 0)),
            pl.BlockSpec((B, _TS, D), lambda s: (0, s, 0)),
        ],
        out_specs=pl.BlockSpec((B, _TS, D), lambda s: (0, s, 0)),
        out_shape=jax.ShapeDtypeStruct((B, S, D), jnp.float32),
    )(table, W, bias2d, inputs)


# speedup vs baseline: 1.0966x; 1.0149x over previous
import jax
import jax.numpy as jnp
from jax.experimental import pallas as pl
from jax.experimental.pallas import tpu as pltpu

_TS = 512


def _fused_kernel(table_ref, w_ref, bias_ref, x_ref, o_ref):
    pos = (
        jnp.dot(table_ref[...].astype(jnp.bfloat16),
                w_ref[...].astype(jnp.bfloat16),
                preferred_element_type=jnp.float32)
        + bias_ref[...]
    )
    o_ref[...] = x_ref[...] + pos[None, :, :]


def kernel(inputs, table, W, b):
    B, S, D = inputs.shape
    bias2d = b.reshape(1, D)
    grid = (S // _TS,)
    return pl.pallas_call(
        _fused_kernel,
        grid=grid,
        in_specs=[
            pl.BlockSpec((_TS, D), lambda s: (s, 0)),
            pl.BlockSpec((D, D), lambda s: (0, 0)),
            pl.BlockSpec((1, D), lambda s: (0, 0)),
            pl.BlockSpec((B, _TS, D), lambda s: (0, s, 0)),
        ],
        out_specs=pl.BlockSpec((B, _TS, D), lambda s: (0, s, 0)),
        out_shape=jax.ShapeDtypeStruct((B, S, D), jnp.float32),
    )(table, W, bias2d, inputs)
